# baseline (speedup 1.0000x reference)
import functools

import jax
import jax.numpy as jnp
from jax import lax
from jax.experimental import pallas as pl
from jax.experimental.pallas import tpu as pltpu
from jax.experimental.pallas import tpu_sc as plsc

VOCAB = 1000000
EMBED = 64
BATCH = 16384
BLK = 128

NUM_CORES = 2
NUM_SUBCORES = 16
LANES = 16
NW = NUM_CORES * NUM_SUBCORES
B_PER_W = BATCH // NW
NQ = EMBED // LANES
DEPTH = 6
LOOKAHEAD = 5


def _make_phase1():
    mesh = plsc.VectorSubcoreMesh(core_axis_name="c", subcore_axis_name="s")

    @functools.partial(
        pl.kernel,
        mesh=mesh,
        out_type=(
            jax.ShapeDtypeStruct((BATCH, BLK), jnp.float32),
            jax.ShapeDtypeStruct((BATCH, BLK), jnp.float32),
        ),
        compiler_params=pltpu.CompilerParams(needs_layout_passes=False),
        scratch_types=[
            pltpu.VMEM((B_PER_W,), jnp.int32),
            pltpu.VMEM((B_PER_W,), jnp.int32),
            pltpu.VMEM((B_PER_W // LANES, LANES), jnp.int32),
            pltpu.VMEM((B_PER_W // LANES, LANES), jnp.int32),
            pltpu.VMEM((DEPTH, EMBED, BLK), jnp.float32),
            pltpu.VMEM((DEPTH, EMBED, BLK), jnp.float32),
            pltpu.VMEM((2, LANES, BLK), jnp.float32),
            pltpu.VMEM((2, LANES, BLK), jnp.float32),
            pltpu.SemaphoreType.DMA((DEPTH,)),
            pltpu.SemaphoreType.DMA((DEPTH,)),
            pltpu.SemaphoreType.DMA((2,)),
            pltpu.SemaphoreType.DMA((2,)),
        ],
    )
    def k(s0_hbm, p0_hbm, s1_hbm, p1_hbm, e1t_hbm, e2t_hbm,
          u_hbm, v_hbm, sidx0_v, sidx1_v, pv20, pv21, ring0, ring1,
          rstage0, rstage1, semb0, semb1, sems0, sems1):
        wid = lax.axis_index("s") * NUM_CORES + lax.axis_index("c")
        base = wid * B_PER_W
        lane = lax.iota(jnp.int32, LANES)

        def ext(vec, j):
            return jnp.max(jnp.where(lane == j, vec, 0))

        tabs = []
        for t, (s_hbm, p_hbm, t_hbm, o_hbm, semb, sems, sv, pv, rg, rs) in \
                enumerate((
                (s0_hbm, p0_hbm, e1t_hbm, u_hbm, semb0, sems0, sidx0_v, pv20,
                 ring0, rstage0),
                (s1_hbm, p1_hbm, e2t_hbm, v_hbm, semb1, sems1, sidx1_v, pv21,
                 ring1, rstage1))):
            pltpu.sync_copy(s_hbm.at[pl.ds(base, B_PER_W)], sv)
            pltpu.sync_copy(
                p_hbm.at[pl.ds(wid * (B_PER_W // LANES), B_PER_W // LANES)],
                pv)
            tabs.append(dict(t=t, t_hbm=t_hbm, o_hbm=o_hbm, semb=semb,
                             sems=sems, sv=sv, pv=pv, ring=rg, rstage=rs))

        def fire(tb, col, slot):
            pltpu.async_copy(
                tb["t_hbm"].at[:, pl.ds(pl.multiple_of(col, BLK), BLK)],
                tb["ring"].at[slot], tb["semb"].at[slot])

        def drain_block(tb, slot):
            pltpu.make_async_copy(
                tb["t_hbm"].at[:, pl.ds(0, BLK)], tb["ring"].at[0],
                tb["semb"].at[slot]).wait()

        def drain_scatter(tb, gb):
            pltpu.make_async_copy(
                tb["o_hbm"].at[pl.ds(0, LANES)], tb["rstage"].at[0],
                tb["sems"].at[gb]).wait()

        jf0s = []
        for tb in tabs:
            sv = tb["sv"]
            cur0 = sv[pl.ds(0, LANES)]
            prv0 = plsc.load_gather(sv, [jnp.maximum(lane - 1, 0)])
            new0 = (((cur0 >> 7) != (prv0 >> 7)) | (lane == 0)) & (
                lane < LOOKAHEAD)
            inc0 = plsc.cumsum(new0.astype(jnp.int32))
            pk0 = (cur0 >> 7) | (inc0 << 13)
            jf_prev = jnp.int32(0)
            for j in range(LOOKAHEAD):
                pj = ext(pk0, j)
                jf_j = pj >> 13

                @pl.when(jf_j != jf_prev)
                def _(pj=pj, jf_j=jf_j, tb=tb):
                    fire(tb, (pj & 8191) << 7, (jf_j - 1) % DEPTH)

                jf_prev = jf_j
            jf0s.append(jnp.max(inc0))

        def body(g, carry):
            jf0, ju0, jf1, ju1 = carry
            jfs = [jf0, jf1]
            jus = [ju0, ju1]
            gpos = g * LANES + lane
            fpos = gpos + LOOKAHEAD
            fval = fpos < B_PER_W
            fposc = jnp.minimum(fpos, B_PER_W - 1)
            pks = []
            for tb in tabs:
                sv = tb["sv"]
                gvec = plsc.load_gather(sv, [gpos])
                prv = plsc.load_gather(sv, [jnp.maximum(gpos - 1, 0)])
                newc = ((gvec >> 7) != (prv >> 7)) | (gpos == 0)
                dju = plsc.cumsum(newc.astype(jnp.int32))
                fvec = plsc.load_gather(sv, [fposc])
                fprv = plsc.load_gather(sv, [fposc - 1])
                newf = ((fvec >> 7) != (fprv >> 7)) & fval
                djf = plsc.cumsum(newf.astype(jnp.int32))
                pks.append((gvec & (BLK - 1)) | (dju << 7) | (djf << 12)
                           | ((fvec >> 7) << 17))

            gb = g & 1

            @pl.when(g >= 2)
            def _():
                drain_scatter(tabs[0], gb)
                drain_scatter(tabs[1], gb)

            ju_prev = list(jus)
            jf_prev = list(jfs)
            for j in range(LANES):
                for tb in tabs:
                    t = tb["t"]
                    pa = ext(pks[t], j)
                    jf_j = jfs[t] + ((pa >> 12) & 31)

                    @pl.when(jf_j != jf_prev[t])
                    def _(pa=pa, jf_j=jf_j, tb=tb):
                        fire(tb, (pa >> 17) << 7, (jf_j - 1) % DEPTH)

                    jf_prev[t] = jf_j
                    ju_j = jus[t] + ((pa >> 7) & 31)

                    ju_used = ju_prev[t]
                    ju_prev[t] = ju_j
                    bi = (ju_j - 1) % DEPTH

                    @pl.when(ju_j != ju_used)
                    def _(tb=tb, bi=bi):
                        drain_block(tb, bi)

                    cols = jnp.full((LANES,), pa & (BLK - 1), jnp.int32)
                    for q in range(NQ):
                        rows = q * LANES + lane
                        tb["rstage"][gb, j, pl.ds(q * LANES, LANES)] = (
                            plsc.load_gather(tb["ring"].at[bi], [rows, cols]))

            for tb in tabs:
                pltpu.async_copy(
                    tb["rstage"].at[gb], tb["o_hbm"].at[tb["pv"].at[g]],
                    tb["sems"].at[gb])

            return jf_prev[0], ju_prev[0], jf_prev[1], ju_prev[1]

        lax.fori_loop(0, B_PER_W // LANES, body,
                      (jf0s[0], jnp.int32(0), jf0s[1], jnp.int32(0)))
        for gb in (0, 1):
            drain_scatter(tabs[0], gb)
            drain_scatter(tabs[1], gb)

    return k


def _make_phase2():
    rows = 8192

    def body(u_ref, v_ref, o_ref):
        w = u_ref[:, :EMBED] * v_ref[:, :EMBED]
        o_ref[...] = 1.0 / (1.0 + jnp.exp(-jnp.sum(w, axis=1)))

    return pl.pallas_call(
        body,
        grid=(BATCH // rows,),
        in_specs=[
            pl.BlockSpec((rows, BLK), lambda i: (i, 0)),
            pl.BlockSpec((rows, BLK), lambda i: (i, 0)),
        ],
        out_specs=pl.BlockSpec((rows,), lambda i: (i,)),
        out_shape=jax.ShapeDtypeStruct((BATCH,), jnp.float32),
    )


_phase1 = _make_phase1()
_phase2 = _make_phase2()


@jax.jit
def kernel(X_batch, emb1, emb2):
    idx0 = X_batch[:, 0].astype(jnp.int32)
    idx1 = X_batch[:, 1].astype(jnp.int32)
    pos = lax.iota(jnp.int32, BATCH)
    s0, p0 = lax.sort((idx0, pos), num_keys=1)
    s1, p1 = lax.sort((idx1, pos), num_keys=1)
    p0r = jnp.reshape(p0, (BATCH // LANES, LANES))
    p1r = jnp.reshape(p1, (BATCH // LANES, LANES))
    u, v = _phase1(s0, p0r, s1, p1r, emb1.T, emb2.T)
    return _phase2(u, v)
